# Initial kernel scaffold; baseline (speedup 1.0000x reference)
import jax
import jax.numpy as jnp
from jax.experimental import pallas as pl


def kernel(x, w_d0c0, b_d0c0, w_d0c1, b_d0c1, w_d1c0, b_d1c0, w_d1c1, b_d1c1, w_d2c0, b_d2c0, w_d2c1, b_d2c1, w_u0c0, b_u0c0, w_u0c1, b_u0c1, w_u1c0, b_u1c0, w_u1c1, b_u1c1, head_w, head_b):
    raise NotImplementedError("write your pallas kernel here")


# speedup vs baseline: 4.2420x; 4.2420x over previous
import functools

import jax
import jax.numpy as jnp
from jax import lax
from jax.experimental import pallas as pl
from jax.experimental.pallas import tpu as pltpu

_BF = jnp.bfloat16
_PARAMS = pltpu.CompilerParams(dimension_semantics=("parallel", "arbitrary"))


def _fold_w(p):
    H, C, W = p.shape
    z = jnp.zeros((H, C, 1), p.dtype)
    left = jnp.concatenate([z, p[:, :, : W - 1]], axis=2)
    right = jnp.concatenate([p[:, :, 1:], z], axis=2)
    return jnp.concatenate([left, p, right], axis=1)


def _prep_w(w):
    cout, cin = w.shape[0], w.shape[1]
    return jnp.transpose(w, (2, 3, 0, 4, 1)).reshape(3, 3, cout, 3 * cin).astype(_BF)


def _halo_masks(dl, Dl):
    d = pl.program_id(1)
    m0 = jnp.where(d > 0, 1.0, 0.0).astype(_BF)
    m2 = jnp.where(d < Dl - 1, 1.0, 0.0).astype(_BF)
    return m0, m2


def _conv_rows(H, f, ws, b, dp):
    rows = []
    for h in range(H):
        acc = None
        for kd in range(3):
            fp = f[dp + kd]
            for kh in range(3):
                r = h + kh - 1
                if 0 <= r < H:
                    t = jnp.dot(ws[kd][kh], fp[r], preferred_element_type=jnp.float32)
                    acc = t if acc is None else acc + t
        rows.append(jnp.maximum(acc + b, 0.0))
    return rows


def _conv_pair_kernel(H, Dl, xA_ref, xB_ref, xC_ref, w_ref, b_ref, o_ref):
    m0, m2 = _halo_masks(None, Dl)
    f = [
        _fold_w(xA_ref[0, 1] * m0),
        _fold_w(xB_ref[0, 0]),
        _fold_w(xB_ref[0, 1]),
        _fold_w(xC_ref[0, 0] * m2),
    ]
    ws = [[w_ref[kd, kh] for kh in range(3)] for kd in range(3)]
    b = b_ref[...]
    for dp in range(2):
        for h, row in enumerate(_conv_rows(H, f, ws, b, dp)):
            o_ref[0, dp, h] = row.astype(_BF)


def _conv_pool_kernel(H, Dl, xA_ref, xB_ref, xC_ref, w_ref, b_ref, se_ref, so_ref,
                      o_ref, p_ref):
    m0, m2 = _halo_masks(None, Dl)
    f = [
        _fold_w(xA_ref[0, 1] * m0),
        _fold_w(xB_ref[0, 0]),
        _fold_w(xB_ref[0, 1]),
        _fold_w(xC_ref[0, 0] * m2),
    ]
    ws = [[w_ref[kd, kh] for kh in range(3)] for kd in range(3)]
    b = b_ref[...]
    rows = []
    for dp in range(2):
        drows = []
        for h, row in enumerate(_conv_rows(H, f, ws, b, dp)):
            rb = row.astype(_BF)
            o_ref[0, dp, h] = rb
            drows.append(rb)
        rows.append(drows)
    se = se_ref[...]
    so = so_ref[...]
    for ho in range(H // 2):
        r = jnp.maximum(
            jnp.maximum(rows[0][2 * ho], rows[0][2 * ho + 1]),
            jnp.maximum(rows[1][2 * ho], rows[1][2 * ho + 1]),
        )
        ev = jnp.dot(r, se, preferred_element_type=jnp.float32)
        od = jnp.dot(r, so, preferred_element_type=jnp.float32)
        p_ref[0, 0, ho] = jnp.maximum(ev, od).astype(_BF)


def _conv_head_kernel(H, Dl, xA_ref, xB_ref, xC_ref, w_ref, b_ref, hw_ref, hb_ref,
                      o_ref):
    m0, m2 = _halo_masks(None, Dl)
    f = [
        _fold_w(xA_ref[0, 1] * m0),
        _fold_w(xB_ref[0, 0]),
        _fold_w(xB_ref[0, 1]),
        _fold_w(xC_ref[0, 0] * m2),
    ]
    ws = [[w_ref[kd, kh] for kh in range(3)] for kd in range(3)]
    b = b_ref[...]
    hw = hw_ref[...]
    hb = hb_ref[...]
    for dp in range(2):
        for h, row in enumerate(_conv_rows(H, f, ws, b, dp)):
            z = jnp.dot(hw, row, preferred_element_type=jnp.float32) + hb
            o_ref[0, :, dp, h, :] = 1.0 / (1.0 + jnp.exp(-z))


def _conv_pair(x, w, b, pool=False, head=None):
    N, D, H, C, W = x.shape
    Cout = w.shape[0]
    Dl = D // 2
    wr = _prep_w(w)
    br = b.reshape(Cout, 1)

    xspec = lambda fn: pl.BlockSpec((1, 2, H, C, W), fn)
    in_specs = [
        xspec(lambda n, d: (n, jnp.maximum(d - 1, 0), 0, 0, 0)),
        xspec(lambda n, d: (n, d, 0, 0, 0)),
        xspec(lambda n, d: (n, jnp.minimum(d + 1, Dl - 1), 0, 0, 0)),
        pl.BlockSpec((3, 3, Cout, 3 * C), lambda n, d: (0, 0, 0, 0)),
        pl.BlockSpec((Cout, 1), lambda n, d: (0, 0)),
    ]
    args = [x, x, x, wr, br]

    if head is not None:
        hw, hb = head
        K = hw.shape[0]
        in_specs += [
            pl.BlockSpec((K, Cout), lambda n, d: (0, 0)),
            pl.BlockSpec((K, 1), lambda n, d: (0, 0)),
        ]
        args += [hw, hb.reshape(K, 1)]
        return pl.pallas_call(
            functools.partial(_conv_head_kernel, H, Dl),
            out_shape=jax.ShapeDtypeStruct((N, K, D, H, W), jnp.float32),
            grid_spec=pltpu.PrefetchScalarGridSpec(
                num_scalar_prefetch=0,
                grid=(N, Dl),
                in_specs=in_specs,
                out_specs=pl.BlockSpec((1, K, 2, H, W), lambda n, d: (n, 0, d, 0, 0)),
            ),
            compiler_params=_PARAMS,
        )(*args)

    if pool:
        cols = jnp.arange(W // 2)
        se = (jnp.arange(W)[:, None] == 2 * cols[None, :]).astype(_BF)
        so = (jnp.arange(W)[:, None] == 2 * cols[None, :] + 1).astype(_BF)
        in_specs += [
            pl.BlockSpec((W, W // 2), lambda n, d: (0, 0)),
            pl.BlockSpec((W, W // 2), lambda n, d: (0, 0)),
        ]
        args += [se, so]
        return pl.pallas_call(
            functools.partial(_conv_pool_kernel, H, Dl),
            out_shape=[
                jax.ShapeDtypeStruct((N, D, H, Cout, W), _BF),
                jax.ShapeDtypeStruct((N, Dl, H // 2, Cout, W // 2), _BF),
            ],
            grid_spec=pltpu.PrefetchScalarGridSpec(
                num_scalar_prefetch=0,
                grid=(N, Dl),
                in_specs=in_specs,
                out_specs=[
                    pl.BlockSpec((1, 2, H, Cout, W), lambda n, d: (n, d, 0, 0, 0)),
                    pl.BlockSpec((1, 1, H // 2, Cout, W // 2), lambda n, d: (n, d, 0, 0, 0)),
                ],
            ),
            compiler_params=_PARAMS,
        )(*args)

    return pl.pallas_call(
        functools.partial(_conv_pair_kernel, H, Dl),
        out_shape=jax.ShapeDtypeStruct((N, D, H, Cout, W), _BF),
        grid_spec=pltpu.PrefetchScalarGridSpec(
            num_scalar_prefetch=0,
            grid=(N, Dl),
            in_specs=in_specs,
            out_specs=pl.BlockSpec((1, 2, H, Cout, W), lambda n, d: (n, d, 0, 0, 0)),
        ),
        compiler_params=_PARAMS,
    )(*args)


def _up_conv_kernel(H, Dl, lA_ref, lB_ref, lC_ref, sA_ref, sB_ref, sC_ref,
                    wl_ref, ws_ref, b_ref, e_ref, o_ref):
    m0, m2 = _halo_masks(None, Dl)
    E = e_ref[...]
    dims = (((2,), (0,)), ((), ()))

    def expand(p):
        return lax.dot_general(p, E, dims, preferred_element_type=jnp.float32).astype(_BF)

    f_up = [
        _fold_w(expand(lA_ref[0, 0]) * m0),
        _fold_w(expand(lB_ref[0, 0])),
        _fold_w(expand(lC_ref[0, 0]) * m2),
    ]
    f_sk = [
        _fold_w(sA_ref[0, 1] * m0),
        _fold_w(sB_ref[0, 0]),
        _fold_w(sB_ref[0, 1]),
        _fold_w(sC_ref[0, 0] * m2),
    ]
    wl = [[wl_ref[kd, kh] for kh in range(3)] for kd in range(3)]
    wsk = [[ws_ref[kd, kh] for kh in range(3)] for kd in range(3)]
    b = b_ref[...]
    for dp in range(2):
        lidx = (0, 1, 1) if dp == 0 else (1, 1, 2)
        for h in range(H):
            acc = None
            for kd in range(3):
                fu = f_up[lidx[kd]]
                fs = f_sk[dp + kd]
                for kh in range(3):
                    r = h + kh - 1
                    if 0 <= r < H:
                        t = jnp.dot(wsk[kd][kh], fs[r], preferred_element_type=jnp.float32)
                        t = t + jnp.dot(wl[kd][kh], fu[r // 2], preferred_element_type=jnp.float32)
                        acc = t if acc is None else acc + t
            o_ref[0, dp, h] = jnp.maximum(acc + b, 0.0).astype(_BF)


def _up_conv(low, skip, w, b):
    N, Dl, Hl, Cl, Wl = low.shape
    _, D, H, Cs, W = skip.shape
    Cout = w.shape[0]
    wl = _prep_w(w[:, :Cl])
    wsk = _prep_w(w[:, Cl:])
    br = b.reshape(Cout, 1)
    E = (jnp.arange(Wl)[:, None] == (jnp.arange(W)[None, :] // 2)).astype(_BF)

    lspec = lambda fn: pl.BlockSpec((1, 1, Hl, Cl, Wl), fn)
    sspec = lambda fn: pl.BlockSpec((1, 2, H, Cs, W), fn)
    in_specs = [
        lspec(lambda n, d: (n, jnp.maximum(d - 1, 0), 0, 0, 0)),
        lspec(lambda n, d: (n, d, 0, 0, 0)),
        lspec(lambda n, d: (n, jnp.minimum(d + 1, Dl - 1), 0, 0, 0)),
        sspec(lambda n, d: (n, jnp.maximum(d - 1, 0), 0, 0, 0)),
        sspec(lambda n, d: (n, d, 0, 0, 0)),
        sspec(lambda n, d: (n, jnp.minimum(d + 1, Dl - 1), 0, 0, 0)),
        pl.BlockSpec((3, 3, Cout, 3 * Cl), lambda n, d: (0, 0, 0, 0)),
        pl.BlockSpec((3, 3, Cout, 3 * Cs), lambda n, d: (0, 0, 0, 0)),
        pl.BlockSpec((Cout, 1), lambda n, d: (0, 0)),
        pl.BlockSpec((Wl, W), lambda n, d: (0, 0)),
    ]
    return pl.pallas_call(
        functools.partial(_up_conv_kernel, H, Dl),
        out_shape=jax.ShapeDtypeStruct((N, D, H, Cout, W), _BF),
        grid_spec=pltpu.PrefetchScalarGridSpec(
            num_scalar_prefetch=0,
            grid=(N, Dl),
            in_specs=in_specs,
            out_specs=pl.BlockSpec((1, 2, H, Cout, W), lambda n, d: (n, d, 0, 0, 0)),
        ),
        compiler_params=_PARAMS,
    )(low, low, low, skip, skip, skip, wl, wsk, br, E)


def kernel(x, w_d0c0, b_d0c0, w_d0c1, b_d0c1, w_d1c0, b_d1c0, w_d1c1, b_d1c1,
           w_d2c0, b_d2c0, w_d2c1, b_d2c1, w_u0c0, b_u0c0, w_u0c1, b_u0c1,
           w_u1c0, b_u1c0, w_u1c1, b_u1c1, head_w, head_b):
    xb = jnp.transpose(x, (0, 2, 3, 1, 4)).astype(_BF)
    h = _conv_pair(xb, w_d0c0, b_d0c0)
    s0, h = _conv_pair(h, w_d0c1, b_d0c1, pool=True)
    h = _conv_pair(h, w_d1c0, b_d1c0)
    s1, h = _conv_pair(h, w_d1c1, b_d1c1, pool=True)
    h = _conv_pair(h, w_d2c0, b_d2c0)
    h = _conv_pair(h, w_d2c1, b_d2c1)
    h = _up_conv(h, s1, w_u1c0, b_u1c0)
    h = _conv_pair(h, w_u1c1, b_u1c1)
    h = _up_conv(h, s0, w_u0c0, b_u0c0)
    return _conv_pair(h, w_u0c1, b_u0c1, head=(head_w, head_b))


# speedup vs baseline: 4.3212x; 1.0187x over previous
import functools

import jax
import jax.numpy as jnp
from jax import lax
from jax.experimental import pallas as pl
from jax.experimental.pallas import tpu as pltpu

_BF = jnp.bfloat16
_PARAMS = pltpu.CompilerParams(dimension_semantics=("parallel", "arbitrary"))


def _fold_w(p):
    H, C, W = p.shape
    z = jnp.zeros((H, C, 1), p.dtype)
    left = jnp.concatenate([z, p[:, :, : W - 1]], axis=2)
    right = jnp.concatenate([p[:, :, 1:], z], axis=2)
    return jnp.concatenate([left, p, right], axis=1)


def _prep_w(w):
    cout, cin = w.shape[0], w.shape[1]
    return jnp.transpose(w, (2, 3, 0, 4, 1)).reshape(3, 3, cout, 3 * cin).astype(_BF)


def _halo_masks(dl, Dl):
    d = pl.program_id(1)
    m0 = jnp.where(d > 0, 1.0, 0.0).astype(_BF)
    m2 = jnp.where(d < Dl - 1, 1.0, 0.0).astype(_BF)
    return m0, m2


def _conv_rows(H, f, ws, b):
    accs = [[None] * H for _ in range(2)]
    for kd in range(3):
        for kh in range(3):
            w = ws[kd][kh]
            for dp in range(2):
                fp = f[dp + kd]
                for h in range(H):
                    r = h + kh - 1
                    if 0 <= r < H:
                        t = jnp.dot(w, fp[r], preferred_element_type=jnp.float32)
                        accs[dp][h] = t if accs[dp][h] is None else accs[dp][h] + t
    return [[jnp.maximum(a + b, 0.0) for a in accs[dp]] for dp in range(2)]


def _conv_pair_kernel(H, Dl, xA_ref, xB_ref, xC_ref, w_ref, b_ref, o_ref):
    m0, m2 = _halo_masks(None, Dl)
    f = [
        _fold_w(xA_ref[0, 1] * m0),
        _fold_w(xB_ref[0, 0]),
        _fold_w(xB_ref[0, 1]),
        _fold_w(xC_ref[0, 0] * m2),
    ]
    ws = [[w_ref[kd, kh] for kh in range(3)] for kd in range(3)]
    b = b_ref[...]
    rows = _conv_rows(H, f, ws, b)
    for dp in range(2):
        for h in range(H):
            o_ref[0, dp, h] = rows[dp][h].astype(_BF)


def _conv_pool_kernel(H, Dl, xA_ref, xB_ref, xC_ref, w_ref, b_ref, se_ref, so_ref,
                      o_ref, p_ref):
    m0, m2 = _halo_masks(None, Dl)
    f = [
        _fold_w(xA_ref[0, 1] * m0),
        _fold_w(xB_ref[0, 0]),
        _fold_w(xB_ref[0, 1]),
        _fold_w(xC_ref[0, 0] * m2),
    ]
    ws = [[w_ref[kd, kh] for kh in range(3)] for kd in range(3)]
    b = b_ref[...]
    frows = _conv_rows(H, f, ws, b)
    rows = []
    for dp in range(2):
        drows = []
        for h in range(H):
            rb = frows[dp][h].astype(_BF)
            o_ref[0, dp, h] = rb
            drows.append(rb)
        rows.append(drows)
    se = se_ref[...]
    so = so_ref[...]
    for ho in range(H // 2):
        r = jnp.maximum(
            jnp.maximum(rows[0][2 * ho], rows[0][2 * ho + 1]),
            jnp.maximum(rows[1][2 * ho], rows[1][2 * ho + 1]),
        )
        ev = jnp.dot(r, se, preferred_element_type=jnp.float32)
        od = jnp.dot(r, so, preferred_element_type=jnp.float32)
        p_ref[0, 0, ho] = jnp.maximum(ev, od).astype(_BF)


def _conv_head_kernel(H, Dl, xA_ref, xB_ref, xC_ref, w_ref, b_ref, hw_ref, hb_ref,
                      o_ref):
    m0, m2 = _halo_masks(None, Dl)
    f = [
        _fold_w(xA_ref[0, 1] * m0),
        _fold_w(xB_ref[0, 0]),
        _fold_w(xB_ref[0, 1]),
        _fold_w(xC_ref[0, 0] * m2),
    ]
    ws = [[w_ref[kd, kh] for kh in range(3)] for kd in range(3)]
    b = b_ref[...]
    hw = hw_ref[...]
    hb = hb_ref[...]
    rows = _conv_rows(H, f, ws, b)
    for dp in range(2):
        for h in range(H):
            z = jnp.dot(hw, rows[dp][h], preferred_element_type=jnp.float32) + hb
            o_ref[0, :, dp, h, :] = 1.0 / (1.0 + jnp.exp(-z))


def _conv_pair(x, w, b, pool=False, head=None):
    N, D, H, C, W = x.shape
    Cout = w.shape[0]
    Dl = D // 2
    wr = _prep_w(w)
    br = b.reshape(Cout, 1)

    xspec = lambda fn: pl.BlockSpec((1, 2, H, C, W), fn)
    in_specs = [
        xspec(lambda n, d: (n, jnp.maximum(d - 1, 0), 0, 0, 0)),
        xspec(lambda n, d: (n, d, 0, 0, 0)),
        xspec(lambda n, d: (n, jnp.minimum(d + 1, Dl - 1), 0, 0, 0)),
        pl.BlockSpec((3, 3, Cout, 3 * C), lambda n, d: (0, 0, 0, 0)),
        pl.BlockSpec((Cout, 1), lambda n, d: (0, 0)),
    ]
    args = [x, x, x, wr, br]

    if head is not None:
        hw, hb = head
        K = hw.shape[0]
        in_specs += [
            pl.BlockSpec((K, Cout), lambda n, d: (0, 0)),
            pl.BlockSpec((K, 1), lambda n, d: (0, 0)),
        ]
        args += [hw, hb.reshape(K, 1)]
        return pl.pallas_call(
            functools.partial(_conv_head_kernel, H, Dl),
            out_shape=jax.ShapeDtypeStruct((N, K, D, H, W), jnp.float32),
            grid_spec=pltpu.PrefetchScalarGridSpec(
                num_scalar_prefetch=0,
                grid=(N, Dl),
                in_specs=in_specs,
                out_specs=pl.BlockSpec((1, K, 2, H, W), lambda n, d: (n, 0, d, 0, 0)),
            ),
            compiler_params=_PARAMS,
        )(*args)

    if pool:
        cols = jnp.arange(W // 2)
        se = (jnp.arange(W)[:, None] == 2 * cols[None, :]).astype(_BF)
        so = (jnp.arange(W)[:, None] == 2 * cols[None, :] + 1).astype(_BF)
        in_specs += [
            pl.BlockSpec((W, W // 2), lambda n, d: (0, 0)),
            pl.BlockSpec((W, W // 2), lambda n, d: (0, 0)),
        ]
        args += [se, so]
        return pl.pallas_call(
            functools.partial(_conv_pool_kernel, H, Dl),
            out_shape=[
                jax.ShapeDtypeStruct((N, D, H, Cout, W), _BF),
                jax.ShapeDtypeStruct((N, Dl, H // 2, Cout, W // 2), _BF),
            ],
            grid_spec=pltpu.PrefetchScalarGridSpec(
                num_scalar_prefetch=0,
                grid=(N, Dl),
                in_specs=in_specs,
                out_specs=[
                    pl.BlockSpec((1, 2, H, Cout, W), lambda n, d: (n, d, 0, 0, 0)),
                    pl.BlockSpec((1, 1, H // 2, Cout, W // 2), lambda n, d: (n, d, 0, 0, 0)),
                ],
            ),
            compiler_params=_PARAMS,
        )(*args)

    return pl.pallas_call(
        functools.partial(_conv_pair_kernel, H, Dl),
        out_shape=jax.ShapeDtypeStruct((N, D, H, Cout, W), _BF),
        grid_spec=pltpu.PrefetchScalarGridSpec(
            num_scalar_prefetch=0,
            grid=(N, Dl),
            in_specs=in_specs,
            out_specs=pl.BlockSpec((1, 2, H, Cout, W), lambda n, d: (n, d, 0, 0, 0)),
        ),
        compiler_params=_PARAMS,
    )(*args)


def _up_conv_kernel(H, Dl, lA_ref, lB_ref, lC_ref, sA_ref, sB_ref, sC_ref,
                    wl_ref, ws_ref, b_ref, e_ref, o_ref):
    m0, m2 = _halo_masks(None, Dl)
    E = e_ref[...]
    dims = (((2,), (0,)), ((), ()))

    def expand(p):
        return lax.dot_general(p, E, dims, preferred_element_type=jnp.float32).astype(_BF)

    f_up = [
        _fold_w(expand(lA_ref[0, 0]) * m0),
        _fold_w(expand(lB_ref[0, 0])),
        _fold_w(expand(lC_ref[0, 0]) * m2),
    ]
    f_sk = [
        _fold_w(sA_ref[0, 1] * m0),
        _fold_w(sB_ref[0, 0]),
        _fold_w(sB_ref[0, 1]),
        _fold_w(sC_ref[0, 0] * m2),
    ]
    wl = [[wl_ref[kd, kh] for kh in range(3)] for kd in range(3)]
    wsk = [[ws_ref[kd, kh] for kh in range(3)] for kd in range(3)]
    b = b_ref[...]
    lidx = ((0, 1, 1), (1, 1, 2))
    accs = [[None] * H for _ in range(2)]

    def acc_tap(w, rows_by_h):
        for dp in range(2):
            for h in range(H):
                r = rows_by_h[dp][h]
                if r is None:
                    continue
                t = jnp.dot(w, r, preferred_element_type=jnp.float32)
                accs[dp][h] = t if accs[dp][h] is None else accs[dp][h] + t

    for kd in range(3):
        for kh in range(3):
            acc_tap(wsk[kd][kh],
                    [[f_sk[dp + kd][h + kh - 1] if 0 <= h + kh - 1 < H else None
                      for h in range(H)] for dp in range(2)])
            acc_tap(wl[kd][kh],
                    [[f_up[lidx[dp][kd]][(h + kh - 1) // 2] if 0 <= h + kh - 1 < H else None
                      for h in range(H)] for dp in range(2)])
    for dp in range(2):
        for h in range(H):
            o_ref[0, dp, h] = jnp.maximum(accs[dp][h] + b, 0.0).astype(_BF)


def _up_conv(low, skip, w, b):
    N, Dl, Hl, Cl, Wl = low.shape
    _, D, H, Cs, W = skip.shape
    Cout = w.shape[0]
    wl = _prep_w(w[:, :Cl])
    wsk = _prep_w(w[:, Cl:])
    br = b.reshape(Cout, 1)
    E = (jnp.arange(Wl)[:, None] == (jnp.arange(W)[None, :] // 2)).astype(_BF)

    lspec = lambda fn: pl.BlockSpec((1, 1, Hl, Cl, Wl), fn)
    sspec = lambda fn: pl.BlockSpec((1, 2, H, Cs, W), fn)
    in_specs = [
        lspec(lambda n, d: (n, jnp.maximum(d - 1, 0), 0, 0, 0)),
        lspec(lambda n, d: (n, d, 0, 0, 0)),
        lspec(lambda n, d: (n, jnp.minimum(d + 1, Dl - 1), 0, 0, 0)),
        sspec(lambda n, d: (n, jnp.maximum(d - 1, 0), 0, 0, 0)),
        sspec(lambda n, d: (n, d, 0, 0, 0)),
        sspec(lambda n, d: (n, jnp.minimum(d + 1, Dl - 1), 0, 0, 0)),
        pl.BlockSpec((3, 3, Cout, 3 * Cl), lambda n, d: (0, 0, 0, 0)),
        pl.BlockSpec((3, 3, Cout, 3 * Cs), lambda n, d: (0, 0, 0, 0)),
        pl.BlockSpec((Cout, 1), lambda n, d: (0, 0)),
        pl.BlockSpec((Wl, W), lambda n, d: (0, 0)),
    ]
    return pl.pallas_call(
        functools.partial(_up_conv_kernel, H, Dl),
        out_shape=jax.ShapeDtypeStruct((N, D, H, Cout, W), _BF),
        grid_spec=pltpu.PrefetchScalarGridSpec(
            num_scalar_prefetch=0,
            grid=(N, Dl),
            in_specs=in_specs,
            out_specs=pl.BlockSpec((1, 2, H, Cout, W), lambda n, d: (n, d, 0, 0, 0)),
        ),
        compiler_params=_PARAMS,
    )(low, low, low, skip, skip, skip, wl, wsk, br, E)


def kernel(x, w_d0c0, b_d0c0, w_d0c1, b_d0c1, w_d1c0, b_d1c0, w_d1c1, b_d1c1,
           w_d2c0, b_d2c0, w_d2c1, b_d2c1, w_u0c0, b_u0c0, w_u0c1, b_u0c1,
           w_u1c0, b_u1c0, w_u1c1, b_u1c1, head_w, head_b):
    xb = jnp.transpose(x, (0, 2, 3, 1, 4)).astype(_BF)
    h = _conv_pair(xb, w_d0c0, b_d0c0)
    s0, h = _conv_pair(h, w_d0c1, b_d0c1, pool=True)
    h = _conv_pair(h, w_d1c0, b_d1c0)
    s1, h = _conv_pair(h, w_d1c1, b_d1c1, pool=True)
    h = _conv_pair(h, w_d2c0, b_d2c0)
    h = _conv_pair(h, w_d2c1, b_d2c1)
    h = _up_conv(h, s1, w_u1c0, b_u1c0)
    h = _conv_pair(h, w_u1c1, b_u1c1)
    h = _up_conv(h, s0, w_u0c0, b_u0c0)
    return _conv_pair(h, w_u0c1, b_u0c1, head=(head_w, head_b))


# speedup vs baseline: 4.3307x; 1.0022x over previous
import functools

import jax
import jax.numpy as jnp
from jax import lax
from jax.experimental import pallas as pl
from jax.experimental.pallas import tpu as pltpu

_BF = jnp.bfloat16
_PARAMS = pltpu.CompilerParams(dimension_semantics=("parallel", "arbitrary"))


def _fold_w(p):
    H, C, W = p.shape
    z = jnp.zeros((H, C, 1), p.dtype)
    left = jnp.concatenate([z, p[:, :, : W - 1]], axis=2)
    right = jnp.concatenate([p[:, :, 1:], z], axis=2)
    return jnp.concatenate([left, p, right], axis=1)


def _prep_w(w):
    cout, cin = w.shape[0], w.shape[1]
    return jnp.transpose(w, (2, 3, 0, 4, 1)).reshape(3, 3, cout, 3 * cin).astype(_BF)


def _halo_masks(dl, Dl):
    d = pl.program_id(1)
    m0 = jnp.where(d > 0, 1.0, 0.0).astype(_BF)
    m2 = jnp.where(d < Dl - 1, 1.0, 0.0).astype(_BF)
    return m0, m2


def _conv_rows(H, f, ws, b):
    accs = [[None] * H for _ in range(2)]
    for kd in range(3):
        for kh in range(3):
            w = ws[kd][kh]
            for dp in range(2):
                fp = f[dp + kd]
                for h in range(H):
                    r = h + kh - 1
                    if 0 <= r < H:
                        t = jnp.dot(w, fp[r], preferred_element_type=jnp.float32)
                        accs[dp][h] = t if accs[dp][h] is None else accs[dp][h] + t
    return [[jnp.maximum(a + b, 0.0) for a in accs[dp]] for dp in range(2)]


def _conv_pair_kernel(H, Dl, xA_ref, xB_ref, xC_ref, w_ref, b_ref, o_ref):
    m0, m2 = _halo_masks(None, Dl)
    f = [
        _fold_w(xA_ref[0, 1] * m0),
        _fold_w(xB_ref[0, 0]),
        _fold_w(xB_ref[0, 1]),
        _fold_w(xC_ref[0, 0] * m2),
    ]
    ws = [[w_ref[kd, kh] for kh in range(3)] for kd in range(3)]
    b = b_ref[...]
    rows = _conv_rows(H, f, ws, b)
    for dp in range(2):
        for h in range(H):
            o_ref[0, dp, h] = rows[dp][h].astype(_BF)


def _conv_pool_kernel(H, Dl, xA_ref, xB_ref, xC_ref, w_ref, b_ref, se_ref, so_ref,
                      o_ref, p_ref):
    m0, m2 = _halo_masks(None, Dl)
    f = [
        _fold_w(xA_ref[0, 1] * m0),
        _fold_w(xB_ref[0, 0]),
        _fold_w(xB_ref[0, 1]),
        _fold_w(xC_ref[0, 0] * m2),
    ]
    ws = [[w_ref[kd, kh] for kh in range(3)] for kd in range(3)]
    b = b_ref[...]
    frows = _conv_rows(H, f, ws, b)
    rows = []
    for dp in range(2):
        drows = []
        for h in range(H):
            rb = frows[dp][h].astype(_BF)
            o_ref[0, dp, h] = rb
            drows.append(rb)
        rows.append(drows)
    se = se_ref[...]
    so = so_ref[...]
    for ho in range(H // 2):
        r = jnp.maximum(
            jnp.maximum(rows[0][2 * ho], rows[0][2 * ho + 1]),
            jnp.maximum(rows[1][2 * ho], rows[1][2 * ho + 1]),
        )
        ev = jnp.dot(r, se, preferred_element_type=jnp.float32)
        od = jnp.dot(r, so, preferred_element_type=jnp.float32)
        p_ref[0, 0, ho] = jnp.maximum(ev, od).astype(_BF)


def _conv_head_kernel(H, Dl, xA_ref, xB_ref, xC_ref, w_ref, b_ref, hw_ref, hb_ref,
                      o_ref):
    m0, m2 = _halo_masks(None, Dl)
    f = [
        _fold_w(xA_ref[0, 1] * m0),
        _fold_w(xB_ref[0, 0]),
        _fold_w(xB_ref[0, 1]),
        _fold_w(xC_ref[0, 0] * m2),
    ]
    ws = [[w_ref[kd, kh] for kh in range(3)] for kd in range(3)]
    b = b_ref[...]
    hw = hw_ref[...]
    hb = hb_ref[...]
    rows = _conv_rows(H, f, ws, b)
    for dp in range(2):
        for h in range(H):
            z = jnp.dot(hw, rows[dp][h], preferred_element_type=jnp.float32) + hb
            o_ref[0, :, dp, h, :] = 1.0 / (1.0 + jnp.exp(-z))


def _conv_pair(x, w, b, pool=False, head=None):
    N, D, H, C, W = x.shape
    Cout = w.shape[0]
    Dl = D // 2
    wr = _prep_w(w)
    br = b.reshape(Cout, 1)

    xspec = lambda fn: pl.BlockSpec((1, 2, H, C, W), fn)
    in_specs = [
        xspec(lambda n, d: (n, jnp.maximum(d - 1, 0), 0, 0, 0)),
        xspec(lambda n, d: (n, d, 0, 0, 0)),
        xspec(lambda n, d: (n, jnp.minimum(d + 1, Dl - 1), 0, 0, 0)),
        pl.BlockSpec((3, 3, Cout, 3 * C), lambda n, d: (0, 0, 0, 0)),
        pl.BlockSpec((Cout, 1), lambda n, d: (0, 0)),
    ]
    args = [x, x, x, wr, br]

    if head is not None:
        hw, hb = head
        K = hw.shape[0]
        in_specs += [
            pl.BlockSpec((K, Cout), lambda n, d: (0, 0)),
            pl.BlockSpec((K, 1), lambda n, d: (0, 0)),
        ]
        args += [hw, hb.reshape(K, 1)]
        return pl.pallas_call(
            functools.partial(_conv_head_kernel, H, Dl),
            out_shape=jax.ShapeDtypeStruct((N, K, D, H, W), jnp.float32),
            grid_spec=pltpu.PrefetchScalarGridSpec(
                num_scalar_prefetch=0,
                grid=(N, Dl),
                in_specs=in_specs,
                out_specs=pl.BlockSpec((1, K, 2, H, W), lambda n, d: (n, 0, d, 0, 0)),
            ),
            compiler_params=_PARAMS,
        )(*args)

    if pool:
        cols = jnp.arange(W // 2)
        se = (jnp.arange(W)[:, None] == 2 * cols[None, :]).astype(_BF)
        so = (jnp.arange(W)[:, None] == 2 * cols[None, :] + 1).astype(_BF)
        in_specs += [
            pl.BlockSpec((W, W // 2), lambda n, d: (0, 0)),
            pl.BlockSpec((W, W // 2), lambda n, d: (0, 0)),
        ]
        args += [se, so]
        return pl.pallas_call(
            functools.partial(_conv_pool_kernel, H, Dl),
            out_shape=[
                jax.ShapeDtypeStruct((N, D, H, Cout, W), _BF),
                jax.ShapeDtypeStruct((N, Dl, H // 2, Cout, W // 2), _BF),
            ],
            grid_spec=pltpu.PrefetchScalarGridSpec(
                num_scalar_prefetch=0,
                grid=(N, Dl),
                in_specs=in_specs,
                out_specs=[
                    pl.BlockSpec((1, 2, H, Cout, W), lambda n, d: (n, d, 0, 0, 0)),
                    pl.BlockSpec((1, 1, H // 2, Cout, W // 2), lambda n, d: (n, d, 0, 0, 0)),
                ],
            ),
            compiler_params=_PARAMS,
        )(*args)

    return pl.pallas_call(
        functools.partial(_conv_pair_kernel, H, Dl),
        out_shape=jax.ShapeDtypeStruct((N, D, H, Cout, W), _BF),
        grid_spec=pltpu.PrefetchScalarGridSpec(
            num_scalar_prefetch=0,
            grid=(N, Dl),
            in_specs=in_specs,
            out_specs=pl.BlockSpec((1, 2, H, Cout, W), lambda n, d: (n, d, 0, 0, 0)),
        ),
        compiler_params=_PARAMS,
    )(*args)


def _up_conv_kernel(H, Dl, lA_ref, lB_ref, lC_ref, sA_ref, sB_ref, sC_ref,
                    wl_ref, ws_ref, b_ref, e_ref, o_ref):
    m0, m2 = _halo_masks(None, Dl)
    E = e_ref[...]
    dims = (((2,), (0,)), ((), ()))

    def expand(p):
        return lax.dot_general(p, E, dims, preferred_element_type=jnp.float32).astype(_BF)

    f_up = [
        _fold_w(expand(lA_ref[0, 0]) * m0),
        _fold_w(expand(lB_ref[0, 0])),
        _fold_w(expand(lC_ref[0, 0]) * m2),
    ]
    f_sk = [
        _fold_w(sA_ref[0, 1] * m0),
        _fold_w(sB_ref[0, 0]),
        _fold_w(sB_ref[0, 1]),
        _fold_w(sC_ref[0, 0] * m2),
    ]
    wl = [[wl_ref[kd, kh] for kh in range(3)] for kd in range(3)]
    wsk = [[ws_ref[kd, kh] for kh in range(3)] for kd in range(3)]
    b = b_ref[...]
    lidx = ((0, 1, 1), (1, 1, 2))
    accs = [[None] * H for _ in range(2)]

    def add(dp, h, t):
        accs[dp][h] = t if accs[dp][h] is None else accs[dp][h] + t

    for kd in range(3):
        for kh in range(3):
            w = wsk[kd][kh]
            for dp in range(2):
                fs = f_sk[dp + kd]
                for h in range(H):
                    r = h + kh - 1
                    if 0 <= r < H:
                        add(dp, h, jnp.dot(w, fs[r], preferred_element_type=jnp.float32))
            w = wl[kd][kh]
            cache = {}
            for dp in range(2):
                li = lidx[dp][kd]
                for h in range(H):
                    r = h + kh - 1
                    if 0 <= r < H:
                        key = (li, r // 2)
                        if key not in cache:
                            cache[key] = jnp.dot(w, f_up[li][r // 2],
                                                 preferred_element_type=jnp.float32)
                        add(dp, h, cache[key])
    for dp in range(2):
        for h in range(H):
            o_ref[0, dp, h] = jnp.maximum(accs[dp][h] + b, 0.0).astype(_BF)


def _up_conv(low, skip, w, b):
    N, Dl, Hl, Cl, Wl = low.shape
    _, D, H, Cs, W = skip.shape
    Cout = w.shape[0]
    wl = _prep_w(w[:, :Cl])
    wsk = _prep_w(w[:, Cl:])
    br = b.reshape(Cout, 1)
    E = (jnp.arange(Wl)[:, None] == (jnp.arange(W)[None, :] // 2)).astype(_BF)

    lspec = lambda fn: pl.BlockSpec((1, 1, Hl, Cl, Wl), fn)
    sspec = lambda fn: pl.BlockSpec((1, 2, H, Cs, W), fn)
    in_specs = [
        lspec(lambda n, d: (n, jnp.maximum(d - 1, 0), 0, 0, 0)),
        lspec(lambda n, d: (n, d, 0, 0, 0)),
        lspec(lambda n, d: (n, jnp.minimum(d + 1, Dl - 1), 0, 0, 0)),
        sspec(lambda n, d: (n, jnp.maximum(d - 1, 0), 0, 0, 0)),
        sspec(lambda n, d: (n, d, 0, 0, 0)),
        sspec(lambda n, d: (n, jnp.minimum(d + 1, Dl - 1), 0, 0, 0)),
        pl.BlockSpec((3, 3, Cout, 3 * Cl), lambda n, d: (0, 0, 0, 0)),
        pl.BlockSpec((3, 3, Cout, 3 * Cs), lambda n, d: (0, 0, 0, 0)),
        pl.BlockSpec((Cout, 1), lambda n, d: (0, 0)),
        pl.BlockSpec((Wl, W), lambda n, d: (0, 0)),
    ]
    return pl.pallas_call(
        functools.partial(_up_conv_kernel, H, Dl),
        out_shape=jax.ShapeDtypeStruct((N, D, H, Cout, W), _BF),
        grid_spec=pltpu.PrefetchScalarGridSpec(
            num_scalar_prefetch=0,
            grid=(N, Dl),
            in_specs=in_specs,
            out_specs=pl.BlockSpec((1, 2, H, Cout, W), lambda n, d: (n, d, 0, 0, 0)),
        ),
        compiler_params=_PARAMS,
    )(low, low, low, skip, skip, skip, wl, wsk, br, E)


def kernel(x, w_d0c0, b_d0c0, w_d0c1, b_d0c1, w_d1c0, b_d1c0, w_d1c1, b_d1c1,
           w_d2c0, b_d2c0, w_d2c1, b_d2c1, w_u0c0, b_u0c0, w_u0c1, b_u0c1,
           w_u1c0, b_u1c0, w_u1c1, b_u1c1, head_w, head_b):
    xb = jnp.transpose(x, (0, 2, 3, 1, 4)).astype(_BF)
    h = _conv_pair(xb, w_d0c0, b_d0c0)
    s0, h = _conv_pair(h, w_d0c1, b_d0c1, pool=True)
    h = _conv_pair(h, w_d1c0, b_d1c0)
    s1, h = _conv_pair(h, w_d1c1, b_d1c1, pool=True)
    h = _conv_pair(h, w_d2c0, b_d2c0)
    h = _conv_pair(h, w_d2c1, b_d2c1)
    h = _up_conv(h, s1, w_u1c0, b_u1c0)
    h = _conv_pair(h, w_u1c1, b_u1c1)
    h = _up_conv(h, s0, w_u0c0, b_u0c0)
    return _conv_pair(h, w_u0c1, b_u0c1, head=(head_w, head_b))


# speedup vs baseline: 4.5442x; 1.0493x over previous
import functools

import jax
import jax.numpy as jnp
from jax import lax
from jax.experimental import pallas as pl
from jax.experimental.pallas import tpu as pltpu

_BF = jnp.bfloat16
_PARAMS = pltpu.CompilerParams(dimension_semantics=("parallel", "arbitrary"))


def _fold_w(p):
    H, C, W = p.shape
    z = jnp.zeros((H, C, 1), p.dtype)
    left = jnp.concatenate([z, p[:, :, : W - 1]], axis=2)
    right = jnp.concatenate([p[:, :, 1:], z], axis=2)
    return jnp.concatenate([left, p, right], axis=1)


def _prep_w(w):
    cout, cin = w.shape[0], w.shape[1]
    return jnp.transpose(w, (2, 3, 0, 4, 1)).reshape(3, 3, cout, 3 * cin).astype(_BF)


def _halo_masks(Dg):
    d = pl.program_id(1)
    m0 = jnp.where(d > 0, 1.0, 0.0).astype(_BF)
    m2 = jnp.where(d < Dg - 1, 1.0, 0.0).astype(_BF)
    return m0, m2


def _conv_rows(H, f, ws, b, P):
    accs = [[None] * H for _ in range(P)]
    for kd in range(3):
        for kh in range(3):
            w = ws[kd][kh]
            for dp in range(P):
                fp = f[dp + kd]
                for h in range(H):
                    r = h + kh - 1
                    if 0 <= r < H:
                        t = jnp.dot(w, fp[r], preferred_element_type=jnp.float32)
                        accs[dp][h] = t if accs[dp][h] is None else accs[dp][h] + t
    return [[jnp.maximum(a + b, 0.0) for a in accs[dp]] for dp in range(P)]


def _quad_folds(xA_ref, xB0_ref, xB1_ref, xC_ref, Dq):
    m0, m2 = _halo_masks(Dq)
    return [
        _fold_w(xA_ref[0, 1] * m0),
        _fold_w(xB0_ref[0, 0]),
        _fold_w(xB0_ref[0, 1]),
        _fold_w(xB1_ref[0, 0]),
        _fold_w(xB1_ref[0, 1]),
        _fold_w(xC_ref[0, 0] * m2),
    ]


def _conv_quad_kernel(H, Dq, xA_ref, xB0_ref, xB1_ref, xC_ref, w_ref, b_ref, o_ref):
    f = _quad_folds(xA_ref, xB0_ref, xB1_ref, xC_ref, Dq)
    ws = [[w_ref[kd, kh] for kh in range(3)] for kd in range(3)]
    rows = _conv_rows(H, f, ws, b_ref[...], 4)
    for dp in range(4):
        for h in range(H):
            o_ref[0, dp, h] = rows[dp][h].astype(_BF)


def _conv_pool_kernel(H, Dq, xA_ref, xB0_ref, xB1_ref, xC_ref, w_ref, b_ref,
                      se_ref, so_ref, o_ref, p_ref):
    f = _quad_folds(xA_ref, xB0_ref, xB1_ref, xC_ref, Dq)
    ws = [[w_ref[kd, kh] for kh in range(3)] for kd in range(3)]
    frows = _conv_rows(H, f, ws, b_ref[...], 4)
    rows = []
    for dp in range(4):
        drows = []
        for h in range(H):
            rb = frows[dp][h].astype(_BF)
            o_ref[0, dp, h] = rb
            drows.append(rb)
        rows.append(drows)
    se = se_ref[...]
    so = so_ref[...]
    for pq in range(2):
        for ho in range(H // 2):
            r = jnp.maximum(
                jnp.maximum(rows[2 * pq][2 * ho], rows[2 * pq][2 * ho + 1]),
                jnp.maximum(rows[2 * pq + 1][2 * ho], rows[2 * pq + 1][2 * ho + 1]),
            )
            ev = jnp.dot(r, se, preferred_element_type=jnp.float32)
            od = jnp.dot(r, so, preferred_element_type=jnp.float32)
            p_ref[0, pq, ho] = jnp.maximum(ev, od).astype(_BF)


def _conv_head_kernel(H, Dq, xA_ref, xB0_ref, xB1_ref, xC_ref, w_ref, b_ref,
                      hw_ref, hb_ref, o_ref):
    f = _quad_folds(xA_ref, xB0_ref, xB1_ref, xC_ref, Dq)
    ws = [[w_ref[kd, kh] for kh in range(3)] for kd in range(3)]
    hw = hw_ref[...]
    hb = hb_ref[...]
    rows = _conv_rows(H, f, ws, b_ref[...], 4)
    for dp in range(4):
        for h in range(H):
            z = jnp.dot(hw, rows[dp][h], preferred_element_type=jnp.float32) + hb
            o_ref[0, :, dp, h, :] = 1.0 / (1.0 + jnp.exp(-z))


def _conv_pair(x, w, b, pool=False, head=None):
    N, D, H, C, W = x.shape
    Cout = w.shape[0]
    Dh = D // 2
    Dq = D // 4
    wr = _prep_w(w)
    br = b.reshape(Cout, 1)

    xspec = lambda fn: pl.BlockSpec((1, 2, H, C, W), fn)
    in_specs = [
        xspec(lambda n, d: (n, jnp.maximum(2 * d - 1, 0), 0, 0, 0)),
        xspec(lambda n, d: (n, 2 * d, 0, 0, 0)),
        xspec(lambda n, d: (n, 2 * d + 1, 0, 0, 0)),
        xspec(lambda n, d: (n, jnp.minimum(2 * d + 2, Dh - 1), 0, 0, 0)),
        pl.BlockSpec((3, 3, Cout, 3 * C), lambda n, d: (0, 0, 0, 0)),
        pl.BlockSpec((Cout, 1), lambda n, d: (0, 0)),
    ]
    args = [x, x, x, x, wr, br]

    if head is not None:
        hw, hb = head
        K = hw.shape[0]
        in_specs += [
            pl.BlockSpec((K, Cout), lambda n, d: (0, 0)),
            pl.BlockSpec((K, 1), lambda n, d: (0, 0)),
        ]
        args += [hw, hb.reshape(K, 1)]
        return pl.pallas_call(
            functools.partial(_conv_head_kernel, H, Dq),
            out_shape=jax.ShapeDtypeStruct((N, K, D, H, W), jnp.float32),
            grid_spec=pltpu.PrefetchScalarGridSpec(
                num_scalar_prefetch=0,
                grid=(N, Dq),
                in_specs=in_specs,
                out_specs=pl.BlockSpec((1, K, 4, H, W), lambda n, d: (n, 0, d, 0, 0)),
            ),
            compiler_params=_PARAMS,
        )(*args)

    if pool:
        cols = jnp.arange(W // 2)
        se = (jnp.arange(W)[:, None] == 2 * cols[None, :]).astype(_BF)
        so = (jnp.arange(W)[:, None] == 2 * cols[None, :] + 1).astype(_BF)
        in_specs += [
            pl.BlockSpec((W, W // 2), lambda n, d: (0, 0)),
            pl.BlockSpec((W, W // 2), lambda n, d: (0, 0)),
        ]
        args += [se, so]
        return pl.pallas_call(
            functools.partial(_conv_pool_kernel, H, Dq),
            out_shape=[
                jax.ShapeDtypeStruct((N, D, H, Cout, W), _BF),
                jax.ShapeDtypeStruct((N, D // 2, H // 2, Cout, W // 2), _BF),
            ],
            grid_spec=pltpu.PrefetchScalarGridSpec(
                num_scalar_prefetch=0,
                grid=(N, Dq),
                in_specs=in_specs,
                out_specs=[
                    pl.BlockSpec((1, 4, H, Cout, W), lambda n, d: (n, d, 0, 0, 0)),
                    pl.BlockSpec((1, 2, H // 2, Cout, W // 2), lambda n, d: (n, d, 0, 0, 0)),
                ],
            ),
            compiler_params=_PARAMS,
        )(*args)

    return pl.pallas_call(
        functools.partial(_conv_quad_kernel, H, Dq),
        out_shape=jax.ShapeDtypeStruct((N, D, H, Cout, W), _BF),
        grid_spec=pltpu.PrefetchScalarGridSpec(
            num_scalar_prefetch=0,
            grid=(N, Dq),
            in_specs=in_specs,
            out_specs=pl.BlockSpec((1, 4, H, Cout, W), lambda n, d: (n, d, 0, 0, 0)),
        ),
        compiler_params=_PARAMS,
    )(*args)


def _up_conv_kernel(H, Dl, lA_ref, lB_ref, lC_ref, sA_ref, sB_ref, sC_ref,
                    wl_ref, ws_ref, b_ref, e_ref, o_ref):
    m0, m2 = _halo_masks(Dl)
    E = e_ref[...]
    dims = (((2,), (0,)), ((), ()))

    def expand(p):
        return lax.dot_general(p, E, dims, preferred_element_type=jnp.float32).astype(_BF)

    f_up = [
        _fold_w(expand(lA_ref[0, 0]) * m0),
        _fold_w(expand(lB_ref[0, 0])),
        _fold_w(expand(lC_ref[0, 0]) * m2),
    ]
    f_sk = [
        _fold_w(sA_ref[0, 1] * m0),
        _fold_w(sB_ref[0, 0]),
        _fold_w(sB_ref[0, 1]),
        _fold_w(sC_ref[0, 0] * m2),
    ]
    wl = [[wl_ref[kd, kh] for kh in range(3)] for kd in range(3)]
    wsk = [[ws_ref[kd, kh] for kh in range(3)] for kd in range(3)]
    b = b_ref[...]
    lidx = ((0, 1, 1), (1, 1, 2))
    accs = [[None] * H for _ in range(2)]

    def add(dp, h, t):
        accs[dp][h] = t if accs[dp][h] is None else accs[dp][h] + t

    for kd in range(3):
        for kh in range(3):
            w = wsk[kd][kh]
            for dp in range(2):
                fs = f_sk[dp + kd]
                for h in range(H):
                    r = h + kh - 1
                    if 0 <= r < H:
                        add(dp, h, jnp.dot(w, fs[r], preferred_element_type=jnp.float32))
            w = wl[kd][kh]
            cache = {}
            for dp in range(2):
                li = lidx[dp][kd]
                for h in range(H):
                    r = h + kh - 1
                    if 0 <= r < H:
                        key = (li, r // 2)
                        if key not in cache:
                            cache[key] = jnp.dot(w, f_up[li][r // 2],
                                                 preferred_element_type=jnp.float32)
                        add(dp, h, cache[key])
    for dp in range(2):
        for h in range(H):
            o_ref[0, dp, h] = jnp.maximum(accs[dp][h] + b, 0.0).astype(_BF)


def _up_conv(low, skip, w, b):
    N, Dl, Hl, Cl, Wl = low.shape
    _, D, H, Cs, W = skip.shape
    Cout = w.shape[0]
    wl = _prep_w(w[:, :Cl])
    wsk = _prep_w(w[:, Cl:])
    br = b.reshape(Cout, 1)
    E = (jnp.arange(Wl)[:, None] == (jnp.arange(W)[None, :] // 2)).astype(_BF)

    lspec = lambda fn: pl.BlockSpec((1, 1, Hl, Cl, Wl), fn)
    sspec = lambda fn: pl.BlockSpec((1, 2, H, Cs, W), fn)
    in_specs = [
        lspec(lambda n, d: (n, jnp.maximum(d - 1, 0), 0, 0, 0)),
        lspec(lambda n, d: (n, d, 0, 0, 0)),
        lspec(lambda n, d: (n, jnp.minimum(d + 1, Dl - 1), 0, 0, 0)),
        sspec(lambda n, d: (n, jnp.maximum(d - 1, 0), 0, 0, 0)),
        sspec(lambda n, d: (n, d, 0, 0, 0)),
        sspec(lambda n, d: (n, jnp.minimum(d + 1, Dl - 1), 0, 0, 0)),
        pl.BlockSpec((3, 3, Cout, 3 * Cl), lambda n, d: (0, 0, 0, 0)),
        pl.BlockSpec((3, 3, Cout, 3 * Cs), lambda n, d: (0, 0, 0, 0)),
        pl.BlockSpec((Cout, 1), lambda n, d: (0, 0)),
        pl.BlockSpec((Wl, W), lambda n, d: (0, 0)),
    ]
    return pl.pallas_call(
        functools.partial(_up_conv_kernel, H, Dl),
        out_shape=jax.ShapeDtypeStruct((N, D, H, Cout, W), _BF),
        grid_spec=pltpu.PrefetchScalarGridSpec(
            num_scalar_prefetch=0,
            grid=(N, Dl),
            in_specs=in_specs,
            out_specs=pl.BlockSpec((1, 2, H, Cout, W), lambda n, d: (n, d, 0, 0, 0)),
        ),
        compiler_params=_PARAMS,
    )(low, low, low, skip, skip, skip, wl, wsk, br, E)


def kernel(x, w_d0c0, b_d0c0, w_d0c1, b_d0c1, w_d1c0, b_d1c0, w_d1c1, b_d1c1,
           w_d2c0, b_d2c0, w_d2c1, b_d2c1, w_u0c0, b_u0c0, w_u0c1, b_u0c1,
           w_u1c0, b_u1c0, w_u1c1, b_u1c1, head_w, head_b):
    xb = jnp.transpose(x, (0, 2, 3, 1, 4)).astype(_BF)
    h = _conv_pair(xb, w_d0c0, b_d0c0)
    s0, h = _conv_pair(h, w_d0c1, b_d0c1, pool=True)
    h = _conv_pair(h, w_d1c0, b_d1c0)
    s1, h = _conv_pair(h, w_d1c1, b_d1c1, pool=True)
    h = _conv_pair(h, w_d2c0, b_d2c0)
    h = _conv_pair(h, w_d2c1, b_d2c1)
    h = _up_conv(h, s1, w_u1c0, b_u1c0)
    h = _conv_pair(h, w_u1c1, b_u1c1)
    h = _up_conv(h, s0, w_u0c0, b_u0c0)
    return _conv_pair(h, w_u0c1, b_u0c1, head=(head_w, head_b))


# speedup vs baseline: 4.7905x; 1.0542x over previous
import functools

import jax
import jax.numpy as jnp
from jax import lax
from jax.experimental import pallas as pl
from jax.experimental.pallas import tpu as pltpu

_BF = jnp.bfloat16
_PARAMS = pltpu.CompilerParams(dimension_semantics=("parallel", "arbitrary"))


def _fold_w(p):
    H, C, W = p.shape
    z = jnp.zeros((H, C, 1), p.dtype)
    left = jnp.concatenate([z, p[:, :, : W - 1]], axis=2)
    right = jnp.concatenate([p[:, :, 1:], z], axis=2)
    return jnp.concatenate([left, p, right], axis=1)


def _prep_w(w):
    cout, cin = w.shape[0], w.shape[1]
    return jnp.transpose(w, (2, 3, 0, 4, 1)).reshape(3, 3, cout, 3 * cin).astype(_BF)


def _halo_masks(Dg):
    d = pl.program_id(1)
    m0 = jnp.where(d > 0, 1.0, 0.0).astype(_BF)
    m2 = jnp.where(d < Dg - 1, 1.0, 0.0).astype(_BF)
    return m0, m2


def _conv_rows(H, f, ws, b, P):
    accs = [[None] * H for _ in range(P)]
    for kd in range(3):
        for kh in range(3):
            w = ws[kd][kh]
            for dp in range(P):
                fp = f[dp + kd]
                for h in range(H):
                    r = h + kh - 1
                    if 0 <= r < H:
                        t = jnp.dot(w, fp[r], preferred_element_type=jnp.float32)
                        accs[dp][h] = t if accs[dp][h] is None else accs[dp][h] + t
    return [[jnp.maximum(a + b, 0.0) for a in accs[dp]] for dp in range(P)]


def _quad_folds(xA_ref, xB0_ref, xB1_ref, xC_ref, Dq):
    m0, m2 = _halo_masks(Dq)
    return [
        _fold_w(xA_ref[0, 1] * m0),
        _fold_w(xB0_ref[0, 0]),
        _fold_w(xB0_ref[0, 1]),
        _fold_w(xB1_ref[0, 0]),
        _fold_w(xB1_ref[0, 1]),
        _fold_w(xC_ref[0, 0] * m2),
    ]


def _conv_quad_kernel(H, Dq, xA_ref, xB0_ref, xB1_ref, xC_ref, w_ref, b_ref, o_ref):
    f = _quad_folds(xA_ref, xB0_ref, xB1_ref, xC_ref, Dq)
    ws = [[w_ref[kd, kh] for kh in range(3)] for kd in range(3)]
    rows = _conv_rows(H, f, ws, b_ref[...], 4)
    for dp in range(4):
        for h in range(H):
            o_ref[0, dp, h] = rows[dp][h].astype(_BF)


def _conv_pool_kernel(H, Dq, xA_ref, xB0_ref, xB1_ref, xC_ref, w_ref, b_ref,
                      se_ref, so_ref, o_ref, p_ref):
    f = _quad_folds(xA_ref, xB0_ref, xB1_ref, xC_ref, Dq)
    ws = [[w_ref[kd, kh] for kh in range(3)] for kd in range(3)]
    frows = _conv_rows(H, f, ws, b_ref[...], 4)
    rows = []
    for dp in range(4):
        drows = []
        for h in range(H):
            rb = frows[dp][h].astype(_BF)
            o_ref[0, dp, h] = rb
            drows.append(rb)
        rows.append(drows)
    se = se_ref[...]
    so = so_ref[...]
    for pq in range(2):
        for ho in range(H // 2):
            r = jnp.maximum(
                jnp.maximum(rows[2 * pq][2 * ho], rows[2 * pq][2 * ho + 1]),
                jnp.maximum(rows[2 * pq + 1][2 * ho], rows[2 * pq + 1][2 * ho + 1]),
            )
            ev = jnp.dot(r, se, preferred_element_type=jnp.float32)
            od = jnp.dot(r, so, preferred_element_type=jnp.float32)
            p_ref[0, pq, ho] = jnp.maximum(ev, od).astype(_BF)


def _conv_head_kernel(H, Dq, xA_ref, xB0_ref, xB1_ref, xC_ref, w_ref, b_ref,
                      hw_ref, hb_ref, o_ref):
    f = _quad_folds(xA_ref, xB0_ref, xB1_ref, xC_ref, Dq)
    ws = [[w_ref[kd, kh] for kh in range(3)] for kd in range(3)]
    hw = hw_ref[...]
    hb = hb_ref[...]
    rows = _conv_rows(H, f, ws, b_ref[...], 4)
    for dp in range(4):
        for h in range(H):
            z = jnp.dot(hw, rows[dp][h], preferred_element_type=jnp.float32) + hb
            o_ref[0, :, dp, h, :] = 1.0 / (1.0 + jnp.exp(-z))


def _conv_pair(x, w, b, pool=False, head=None):
    N, D, H, C, W = x.shape
    Cout = w.shape[0]
    Dh = D // 2
    Dq = D // 4
    wr = _prep_w(w)
    br = b.reshape(Cout, 1)

    xspec = lambda fn: pl.BlockSpec((1, 2, H, C, W), fn)
    in_specs = [
        xspec(lambda n, d: (n, jnp.maximum(2 * d - 1, 0), 0, 0, 0)),
        xspec(lambda n, d: (n, 2 * d, 0, 0, 0)),
        xspec(lambda n, d: (n, 2 * d + 1, 0, 0, 0)),
        xspec(lambda n, d: (n, jnp.minimum(2 * d + 2, Dh - 1), 0, 0, 0)),
        pl.BlockSpec((3, 3, Cout, 3 * C), lambda n, d: (0, 0, 0, 0)),
        pl.BlockSpec((Cout, 1), lambda n, d: (0, 0)),
    ]
    args = [x, x, x, x, wr, br]

    if head is not None:
        hw, hb = head
        K = hw.shape[0]
        in_specs += [
            pl.BlockSpec((K, Cout), lambda n, d: (0, 0)),
            pl.BlockSpec((K, 1), lambda n, d: (0, 0)),
        ]
        args += [hw, hb.reshape(K, 1)]
        return pl.pallas_call(
            functools.partial(_conv_head_kernel, H, Dq),
            out_shape=jax.ShapeDtypeStruct((N, K, D, H, W), jnp.float32),
            grid_spec=pltpu.PrefetchScalarGridSpec(
                num_scalar_prefetch=0,
                grid=(N, Dq),
                in_specs=in_specs,
                out_specs=pl.BlockSpec((1, K, 4, H, W), lambda n, d: (n, 0, d, 0, 0)),
            ),
            compiler_params=_PARAMS,
        )(*args)

    if pool:
        cols = jnp.arange(W // 2)
        se = (jnp.arange(W)[:, None] == 2 * cols[None, :]).astype(_BF)
        so = (jnp.arange(W)[:, None] == 2 * cols[None, :] + 1).astype(_BF)
        in_specs += [
            pl.BlockSpec((W, W // 2), lambda n, d: (0, 0)),
            pl.BlockSpec((W, W // 2), lambda n, d: (0, 0)),
        ]
        args += [se, so]
        return pl.pallas_call(
            functools.partial(_conv_pool_kernel, H, Dq),
            out_shape=[
                jax.ShapeDtypeStruct((N, D, H, Cout, W), _BF),
                jax.ShapeDtypeStruct((N, D // 2, H // 2, Cout, W // 2), _BF),
            ],
            grid_spec=pltpu.PrefetchScalarGridSpec(
                num_scalar_prefetch=0,
                grid=(N, Dq),
                in_specs=in_specs,
                out_specs=[
                    pl.BlockSpec((1, 4, H, Cout, W), lambda n, d: (n, d, 0, 0, 0)),
                    pl.BlockSpec((1, 2, H // 2, Cout, W // 2), lambda n, d: (n, d, 0, 0, 0)),
                ],
            ),
            compiler_params=_PARAMS,
        )(*args)

    return pl.pallas_call(
        functools.partial(_conv_quad_kernel, H, Dq),
        out_shape=jax.ShapeDtypeStruct((N, D, H, Cout, W), _BF),
        grid_spec=pltpu.PrefetchScalarGridSpec(
            num_scalar_prefetch=0,
            grid=(N, Dq),
            in_specs=in_specs,
            out_specs=pl.BlockSpec((1, 4, H, Cout, W), lambda n, d: (n, d, 0, 0, 0)),
        ),
        compiler_params=_PARAMS,
    )(*args)


def _up_conv_kernel(H, Dq, lA_ref, lB0_ref, lB1_ref, lC_ref,
                    sA_ref, sB0_ref, sB1_ref, sC_ref,
                    wl_ref, ws_ref, b_ref, e_ref, o_ref):
    m0, m2 = _halo_masks(Dq)
    E = e_ref[...]
    dims = (((2,), (0,)), ((), ()))

    def expand(p):
        return lax.dot_general(p, E, dims, preferred_element_type=jnp.float32).astype(_BF)

    f_up = [
        _fold_w(expand(lA_ref[0, 0]) * m0),
        _fold_w(expand(lB0_ref[0, 0])),
        _fold_w(expand(lB1_ref[0, 0])),
        _fold_w(expand(lC_ref[0, 0]) * m2),
    ]
    f_sk = _quad_folds(sA_ref, sB0_ref, sB1_ref, sC_ref, Dq)
    wl = [[wl_ref[kd, kh] for kh in range(3)] for kd in range(3)]
    wsk = [[ws_ref[kd, kh] for kh in range(3)] for kd in range(3)]
    b = b_ref[...]
    accs = [[None] * H for _ in range(4)]

    def add(dp, h, t):
        accs[dp][h] = t if accs[dp][h] is None else accs[dp][h] + t

    for kd in range(3):
        for kh in range(3):
            w = wsk[kd][kh]
            for dp in range(4):
                fs = f_sk[dp + kd]
                for h in range(H):
                    r = h + kh - 1
                    if 0 <= r < H:
                        add(dp, h, jnp.dot(w, fs[r], preferred_element_type=jnp.float32))
            w = wl[kd][kh]
            cache = {}
            for dp in range(4):
                li = (dp + kd - 1) // 2 + 1
                for h in range(H):
                    r = h + kh - 1
                    if 0 <= r < H:
                        key = (li, r // 2)
                        if key not in cache:
                            cache[key] = jnp.dot(w, f_up[li][r // 2],
                                                 preferred_element_type=jnp.float32)
                        add(dp, h, cache[key])
    for dp in range(4):
        for h in range(H):
            o_ref[0, dp, h] = jnp.maximum(accs[dp][h] + b, 0.0).astype(_BF)


def _up_conv(low, skip, w, b):
    N, Dl, Hl, Cl, Wl = low.shape
    _, D, H, Cs, W = skip.shape
    Cout = w.shape[0]
    Dq = D // 4
    Dh = D // 2
    wl = _prep_w(w[:, :Cl])
    wsk = _prep_w(w[:, Cl:])
    br = b.reshape(Cout, 1)
    E = (jnp.arange(Wl)[:, None] == (jnp.arange(W)[None, :] // 2)).astype(_BF)

    lspec = lambda fn: pl.BlockSpec((1, 1, Hl, Cl, Wl), fn)
    sspec = lambda fn: pl.BlockSpec((1, 2, H, Cs, W), fn)
    in_specs = [
        lspec(lambda n, d: (n, jnp.maximum(2 * d - 1, 0), 0, 0, 0)),
        lspec(lambda n, d: (n, 2 * d, 0, 0, 0)),
        lspec(lambda n, d: (n, 2 * d + 1, 0, 0, 0)),
        lspec(lambda n, d: (n, jnp.minimum(2 * d + 2, Dl - 1), 0, 0, 0)),
        sspec(lambda n, d: (n, jnp.maximum(2 * d - 1, 0), 0, 0, 0)),
        sspec(lambda n, d: (n, 2 * d, 0, 0, 0)),
        sspec(lambda n, d: (n, 2 * d + 1, 0, 0, 0)),
        sspec(lambda n, d: (n, jnp.minimum(2 * d + 2, Dh - 1), 0, 0, 0)),
        pl.BlockSpec((3, 3, Cout, 3 * Cl), lambda n, d: (0, 0, 0, 0)),
        pl.BlockSpec((3, 3, Cout, 3 * Cs), lambda n, d: (0, 0, 0, 0)),
        pl.BlockSpec((Cout, 1), lambda n, d: (0, 0)),
        pl.BlockSpec((Wl, W), lambda n, d: (0, 0)),
    ]
    return pl.pallas_call(
        functools.partial(_up_conv_kernel, H, Dq),
        out_shape=jax.ShapeDtypeStruct((N, D, H, Cout, W), _BF),
        grid_spec=pltpu.PrefetchScalarGridSpec(
            num_scalar_prefetch=0,
            grid=(N, Dq),
            in_specs=in_specs,
            out_specs=pl.BlockSpec((1, 4, H, Cout, W), lambda n, d: (n, d, 0, 0, 0)),
        ),
        compiler_params=_PARAMS,
    )(low, low, low, low, skip, skip, skip, skip, wl, wsk, br, E)


def kernel(x, w_d0c0, b_d0c0, w_d0c1, b_d0c1, w_d1c0, b_d1c0, w_d1c1, b_d1c1,
           w_d2c0, b_d2c0, w_d2c1, b_d2c1, w_u0c0, b_u0c0, w_u0c1, b_u0c1,
           w_u1c0, b_u1c0, w_u1c1, b_u1c1, head_w, head_b):
    xb = jnp.transpose(x, (0, 2, 3, 1, 4)).astype(_BF)
    h = _conv_pair(xb, w_d0c0, b_d0c0)
    s0, h = _conv_pair(h, w_d0c1, b_d0c1, pool=True)
    h = _conv_pair(h, w_d1c0, b_d1c0)
    s1, h = _conv_pair(h, w_d1c1, b_d1c1, pool=True)
    h = _conv_pair(h, w_d2c0, b_d2c0)
    h = _conv_pair(h, w_d2c1, b_d2c1)
    h = _up_conv(h, s1, w_u1c0, b_u1c0)
    h = _conv_pair(h, w_u1c1, b_u1c1)
    h = _up_conv(h, s0, w_u0c0, b_u0c0)
    return _conv_pair(h, w_u0c1, b_u0c1, head=(head_w, head_b))


# speedup vs baseline: 4.9054x; 1.0240x over previous
import functools

import jax
import jax.numpy as jnp
from jax import lax
from jax.experimental import pallas as pl
from jax.experimental.pallas import tpu as pltpu

_BF = jnp.bfloat16
_PARAMS = pltpu.CompilerParams(dimension_semantics=("parallel", "arbitrary"))


def _fold_w(p):
    H, C, W = p.shape
    z = jnp.zeros((H, C, 1), p.dtype)
    left = jnp.concatenate([z, p[:, :, : W - 1]], axis=2)
    right = jnp.concatenate([p[:, :, 1:], z], axis=2)
    return jnp.concatenate([left, p, right], axis=1)


def _prep_w(w):
    cout, cin = w.shape[0], w.shape[1]
    return jnp.transpose(w, (2, 3, 0, 4, 1)).reshape(3, 3, cout, 3 * cin).astype(_BF)


def _halo_masks(Dg):
    d = pl.program_id(1)
    m0 = jnp.where(d > 0, 1.0, 0.0).astype(_BF)
    m2 = jnp.where(d < Dg - 1, 1.0, 0.0).astype(_BF)
    return m0, m2


def _conv_rows(H, f, ws, b, P):
    accs = [[None] * H for _ in range(P)]
    for kd in range(3):
        for kh in range(3):
            w = ws[kd][kh]
            for dp in range(P):
                fp = f[dp + kd]
                for h in range(H):
                    r = h + kh - 1
                    if 0 <= r < H:
                        t = jnp.dot(w, fp[r], preferred_element_type=jnp.float32)
                        accs[dp][h] = t if accs[dp][h] is None else accs[dp][h] + t
    return [[jnp.maximum(a + b, 0.0) for a in accs[dp]] for dp in range(P)]


def _halo_folds(refs, Dg):
    m0, m2 = _halo_masks(Dg)
    f = [_fold_w(refs[0][0, 1] * m0)]
    for br in refs[1:-1]:
        f.append(_fold_w(br[0, 0]))
        f.append(_fold_w(br[0, 1]))
    f.append(_fold_w(refs[-1][0, 0] * m2))
    return f


def _conv_plain_kernel(H, Dg, P, *refs):
    (*xrefs, w_ref, b_ref, o_ref) = refs
    f = _halo_folds(xrefs, Dg)
    ws = [[w_ref[kd, kh] for kh in range(3)] for kd in range(3)]
    rows = _conv_rows(H, f, ws, b_ref[...], P)
    for dp in range(P):
        for h in range(H):
            o_ref[0, dp, h] = rows[dp][h].astype(_BF)


def _conv_pool_kernel(H, Dg, P, *refs):
    (*xrefs, w_ref, b_ref, se_ref, so_ref, o_ref, p_ref) = refs
    f = _halo_folds(xrefs, Dg)
    ws = [[w_ref[kd, kh] for kh in range(3)] for kd in range(3)]
    frows = _conv_rows(H, f, ws, b_ref[...], P)
    rows = []
    for dp in range(P):
        drows = []
        for h in range(H):
            rb = frows[dp][h].astype(_BF)
            o_ref[0, dp, h] = rb
            drows.append(rb)
        rows.append(drows)
    se = se_ref[...]
    so = so_ref[...]
    for pq in range(P // 2):
        for ho in range(H // 2):
            r = jnp.maximum(
                jnp.maximum(rows[2 * pq][2 * ho], rows[2 * pq][2 * ho + 1]),
                jnp.maximum(rows[2 * pq + 1][2 * ho], rows[2 * pq + 1][2 * ho + 1]),
            )
            ev = jnp.dot(r, se, preferred_element_type=jnp.float32)
            od = jnp.dot(r, so, preferred_element_type=jnp.float32)
            p_ref[0, pq, ho] = jnp.maximum(ev, od).astype(_BF)


def _conv_head_kernel(H, Dg, P, *refs):
    (*xrefs, w_ref, b_ref, hw_ref, hb_ref, o_ref) = refs
    f = _halo_folds(xrefs, Dg)
    ws = [[w_ref[kd, kh] for kh in range(3)] for kd in range(3)]
    hw = hw_ref[...]
    hb = hb_ref[...]
    rows = _conv_rows(H, f, ws, b_ref[...], P)
    for dp in range(P):
        for h in range(H):
            z = jnp.dot(hw, rows[dp][h], preferred_element_type=jnp.float32) + hb
            o_ref[0, :, dp, h, :] = 1.0 / (1.0 + jnp.exp(-z))


def _conv_pair(x, w, b, pool=False, head=None):
    N, D, H, C, W = x.shape
    Cout = w.shape[0]
    P = 8 if D % 8 == 0 else 4
    G = P // 2
    Dh = D // 2
    Dq = D // P
    wr = _prep_w(w)
    br = b.reshape(Cout, 1)

    xspec = lambda fn: pl.BlockSpec((1, 2, H, C, W), fn)
    in_specs = [xspec(lambda n, d: (n, jnp.maximum(G * d - 1, 0), 0, 0, 0))]
    in_specs += [xspec(lambda n, d, j=j: (n, G * d + j, 0, 0, 0)) for j in range(G)]
    in_specs += [
        xspec(lambda n, d: (n, jnp.minimum(G * d + G, Dh - 1), 0, 0, 0)),
        pl.BlockSpec((3, 3, Cout, 3 * C), lambda n, d: (0, 0, 0, 0)),
        pl.BlockSpec((Cout, 1), lambda n, d: (0, 0)),
    ]
    args = [x] * (G + 2) + [wr, br]

    if head is not None:
        hw, hb = head
        K = hw.shape[0]
        in_specs += [
            pl.BlockSpec((K, Cout), lambda n, d: (0, 0)),
            pl.BlockSpec((K, 1), lambda n, d: (0, 0)),
        ]
        args += [hw, hb.reshape(K, 1)]
        return pl.pallas_call(
            functools.partial(_conv_head_kernel, H, Dq, P),
            out_shape=jax.ShapeDtypeStruct((N, K, D, H, W), jnp.float32),
            grid_spec=pltpu.PrefetchScalarGridSpec(
                num_scalar_prefetch=0,
                grid=(N, Dq),
                in_specs=in_specs,
                out_specs=pl.BlockSpec((1, K, P, H, W), lambda n, d: (n, 0, d, 0, 0)),
            ),
            compiler_params=_PARAMS,
        )(*args)

    if pool:
        cols = jnp.arange(W // 2)
        se = (jnp.arange(W)[:, None] == 2 * cols[None, :]).astype(_BF)
        so = (jnp.arange(W)[:, None] == 2 * cols[None, :] + 1).astype(_BF)
        in_specs += [
            pl.BlockSpec((W, W // 2), lambda n, d: (0, 0)),
            pl.BlockSpec((W, W // 2), lambda n, d: (0, 0)),
        ]
        args += [se, so]
        return pl.pallas_call(
            functools.partial(_conv_pool_kernel, H, Dq, P),
            out_shape=[
                jax.ShapeDtypeStruct((N, D, H, Cout, W), _BF),
                jax.ShapeDtypeStruct((N, D // 2, H // 2, Cout, W // 2), _BF),
            ],
            grid_spec=pltpu.PrefetchScalarGridSpec(
                num_scalar_prefetch=0,
                grid=(N, Dq),
                in_specs=in_specs,
                out_specs=[
                    pl.BlockSpec((1, P, H, Cout, W), lambda n, d: (n, d, 0, 0, 0)),
                    pl.BlockSpec((1, P // 2, H // 2, Cout, W // 2), lambda n, d: (n, d, 0, 0, 0)),
                ],
            ),
            compiler_params=_PARAMS,
        )(*args)

    return pl.pallas_call(
        functools.partial(_conv_plain_kernel, H, Dq, P),
        out_shape=jax.ShapeDtypeStruct((N, D, H, Cout, W), _BF),
        grid_spec=pltpu.PrefetchScalarGridSpec(
            num_scalar_prefetch=0,
            grid=(N, Dq),
            in_specs=in_specs,
            out_specs=pl.BlockSpec((1, P, H, Cout, W), lambda n, d: (n, d, 0, 0, 0)),
        ),
        compiler_params=_PARAMS,
    )(*args)


def _up_conv_kernel(H, Dq, lA_ref, lB0_ref, lB1_ref, lC_ref,
                    sA_ref, sB0_ref, sB1_ref, sC_ref,
                    wl_ref, ws_ref, b_ref, e_ref, o_ref):
    m0, m2 = _halo_masks(Dq)
    E = e_ref[...]
    dims = (((2,), (0,)), ((), ()))

    def expand(p):
        return lax.dot_general(p, E, dims, preferred_element_type=jnp.float32).astype(_BF)

    f_up = [
        _fold_w(expand(lA_ref[0, 0]) * m0),
        _fold_w(expand(lB0_ref[0, 0])),
        _fold_w(expand(lB1_ref[0, 0])),
        _fold_w(expand(lC_ref[0, 0]) * m2),
    ]
    f_sk = _halo_folds([sA_ref, sB0_ref, sB1_ref, sC_ref], Dq)
    wl = [[wl_ref[kd, kh] for kh in range(3)] for kd in range(3)]
    wsk = [[ws_ref[kd, kh] for kh in range(3)] for kd in range(3)]
    b = b_ref[...]
    accs = [[None] * H for _ in range(4)]

    def add(dp, h, t):
        accs[dp][h] = t if accs[dp][h] is None else accs[dp][h] + t

    for kd in range(3):
        for kh in range(3):
            w = wsk[kd][kh]
            for dp in range(4):
                fs = f_sk[dp + kd]
                for h in range(H):
                    r = h + kh - 1
                    if 0 <= r < H:
                        add(dp, h, jnp.dot(w, fs[r], preferred_element_type=jnp.float32))
            w = wl[kd][kh]
            cache = {}
            for dp in range(4):
                li = (dp + kd - 1) // 2 + 1
                for h in range(H):
                    r = h + kh - 1
                    if 0 <= r < H:
                        key = (li, r // 2)
                        if key not in cache:
                            cache[key] = jnp.dot(w, f_up[li][r // 2],
                                                 preferred_element_type=jnp.float32)
                        add(dp, h, cache[key])
    for dp in range(4):
        for h in range(H):
            o_ref[0, dp, h] = jnp.maximum(accs[dp][h] + b, 0.0).astype(_BF)


def _up_conv(low, skip, w, b):
    N, Dl, Hl, Cl, Wl = low.shape
    _, D, H, Cs, W = skip.shape
    Cout = w.shape[0]
    Dq = D // 4
    Dh = D // 2
    wl = _prep_w(w[:, :Cl])
    wsk = _prep_w(w[:, Cl:])
    br = b.reshape(Cout, 1)
    E = (jnp.arange(Wl)[:, None] == (jnp.arange(W)[None, :] // 2)).astype(_BF)

    lspec = lambda fn: pl.BlockSpec((1, 1, Hl, Cl, Wl), fn)
    sspec = lambda fn: pl.BlockSpec((1, 2, H, Cs, W), fn)
    in_specs = [
        lspec(lambda n, d: (n, jnp.maximum(2 * d - 1, 0), 0, 0, 0)),
        lspec(lambda n, d: (n, 2 * d, 0, 0, 0)),
        lspec(lambda n, d: (n, 2 * d + 1, 0, 0, 0)),
        lspec(lambda n, d: (n, jnp.minimum(2 * d + 2, Dl - 1), 0, 0, 0)),
        sspec(lambda n, d: (n, jnp.maximum(2 * d - 1, 0), 0, 0, 0)),
        sspec(lambda n, d: (n, 2 * d, 0, 0, 0)),
        sspec(lambda n, d: (n, 2 * d + 1, 0, 0, 0)),
        sspec(lambda n, d: (n, jnp.minimum(2 * d + 2, Dh - 1), 0, 0, 0)),
        pl.BlockSpec((3, 3, Cout, 3 * Cl), lambda n, d: (0, 0, 0, 0)),
        pl.BlockSpec((3, 3, Cout, 3 * Cs), lambda n, d: (0, 0, 0, 0)),
        pl.BlockSpec((Cout, 1), lambda n, d: (0, 0)),
        pl.BlockSpec((Wl, W), lambda n, d: (0, 0)),
    ]
    return pl.pallas_call(
        functools.partial(_up_conv_kernel, H, Dq),
        out_shape=jax.ShapeDtypeStruct((N, D, H, Cout, W), _BF),
        grid_spec=pltpu.PrefetchScalarGridSpec(
            num_scalar_prefetch=0,
            grid=(N, Dq),
            in_specs=in_specs,
            out_specs=pl.BlockSpec((1, 4, H, Cout, W), lambda n, d: (n, d, 0, 0, 0)),
        ),
        compiler_params=_PARAMS,
    )(low, low, low, low, skip, skip, skip, skip, wl, wsk, br, E)


def kernel(x, w_d0c0, b_d0c0, w_d0c1, b_d0c1, w_d1c0, b_d1c0, w_d1c1, b_d1c1,
           w_d2c0, b_d2c0, w_d2c1, b_d2c1, w_u0c0, b_u0c0, w_u0c1, b_u0c1,
           w_u1c0, b_u1c0, w_u1c1, b_u1c1, head_w, head_b):
    xb = jnp.transpose(x, (0, 2, 3, 1, 4)).astype(_BF)
    h = _conv_pair(xb, w_d0c0, b_d0c0)
    s0, h = _conv_pair(h, w_d0c1, b_d0c1, pool=True)
    h = _conv_pair(h, w_d1c0, b_d1c0)
    s1, h = _conv_pair(h, w_d1c1, b_d1c1, pool=True)
    h = _conv_pair(h, w_d2c0, b_d2c0)
    h = _conv_pair(h, w_d2c1, b_d2c1)
    h = _up_conv(h, s1, w_u1c0, b_u1c0)
    h = _conv_pair(h, w_u1c1, b_u1c1)
    h = _up_conv(h, s0, w_u0c0, b_u0c0)
    return _conv_pair(h, w_u0c1, b_u0c1, head=(head_w, head_b))


# speedup vs baseline: 5.0617x; 1.0319x over previous
import functools

import jax
import jax.numpy as jnp
from jax import lax
from jax.experimental import pallas as pl
from jax.experimental.pallas import tpu as pltpu

_BF = jnp.bfloat16
_PARAMS = pltpu.CompilerParams(dimension_semantics=("parallel", "arbitrary"))


def _fold_w(p):
    H, C, W = p.shape
    z = jnp.zeros((H, C, 1), p.dtype)
    left = jnp.concatenate([z, p[:, :, : W - 1]], axis=2)
    right = jnp.concatenate([p[:, :, 1:], z], axis=2)
    return jnp.concatenate([left, p, right], axis=1)


def _prep_w(w):
    cout, cin = w.shape[0], w.shape[1]
    return jnp.transpose(w, (2, 3, 0, 4, 1)).reshape(3, 3, cout, 3 * cin).astype(_BF)


def _halo_masks(Dg):
    d = pl.program_id(1)
    m0 = jnp.where(d > 0, 1.0, 0.0).astype(_BF)
    m2 = jnp.where(d < Dg - 1, 1.0, 0.0).astype(_BF)
    return m0, m2


def _conv_rows(H, f, ws, b, P):
    accs = [[None] * H for _ in range(P)]
    for kd in range(3):
        for kh in range(3):
            w = ws[kd][kh]
            for dp in range(P):
                fp = f[dp + kd]
                for h in range(H):
                    r = h + kh - 1
                    if 0 <= r < H:
                        t = jnp.dot(w, fp[r], preferred_element_type=jnp.float32)
                        accs[dp][h] = t if accs[dp][h] is None else accs[dp][h] + t
    return [[jnp.maximum(a + b, 0.0) for a in accs[dp]] for dp in range(P)]


def _halo_folds(refs, Dg):
    m0, m2 = _halo_masks(Dg)
    f = [_fold_w(refs[0][0, 1] * m0)]
    for br in refs[1:-1]:
        f.append(_fold_w(br[0, 0]))
        f.append(_fold_w(br[0, 1]))
    f.append(_fold_w(refs[-1][0, 0] * m2))
    return f


def _conv_plain_kernel(H, Dg, P, *refs):
    (*xrefs, w_ref, b_ref, o_ref) = refs
    f = _halo_folds(xrefs, Dg)
    ws = [[w_ref[kd, kh] for kh in range(3)] for kd in range(3)]
    rows = _conv_rows(H, f, ws, b_ref[...], P)
    for dp in range(P):
        for h in range(H):
            o_ref[0, dp, h] = rows[dp][h].astype(_BF)


def _conv_pool_kernel(H, Dg, P, *refs):
    (*xrefs, w_ref, b_ref, se_ref, so_ref, o_ref, p_ref) = refs
    f = _halo_folds(xrefs, Dg)
    ws = [[w_ref[kd, kh] for kh in range(3)] for kd in range(3)]
    frows = _conv_rows(H, f, ws, b_ref[...], P)
    rows = []
    for dp in range(P):
        drows = []
        for h in range(H):
            rb = frows[dp][h].astype(_BF)
            o_ref[0, dp, h] = rb
            drows.append(rb)
        rows.append(drows)
    se = se_ref[...]
    so = so_ref[...]
    for pq in range(P // 2):
        for ho in range(H // 2):
            r = jnp.maximum(
                jnp.maximum(rows[2 * pq][2 * ho], rows[2 * pq][2 * ho + 1]),
                jnp.maximum(rows[2 * pq + 1][2 * ho], rows[2 * pq + 1][2 * ho + 1]),
            )
            ev = jnp.dot(r, se, preferred_element_type=jnp.float32)
            od = jnp.dot(r, so, preferred_element_type=jnp.float32)
            p_ref[0, pq, ho] = jnp.maximum(ev, od).astype(_BF)


def _conv_head_kernel(H, Dg, P, *refs):
    (*xrefs, w_ref, b_ref, hw_ref, hb_ref, o_ref) = refs
    f = _halo_folds(xrefs, Dg)
    ws = [[w_ref[kd, kh] for kh in range(3)] for kd in range(3)]
    hw = hw_ref[...]
    hb = hb_ref[...]
    rows = _conv_rows(H, f, ws, b_ref[...], P)
    for dp in range(P):
        for h in range(H):
            z = jnp.dot(hw, rows[dp][h], preferred_element_type=jnp.float32) + hb
            o_ref[0, :, dp, h, :] = 1.0 / (1.0 + jnp.exp(-z))


def _conv_pair(x, w, b, pool=False, head=None):
    N, D, H, C, W = x.shape
    Cout = w.shape[0]
    P = 8 if D % 8 == 0 else 4
    G = P // 2
    Dh = D // 2
    Dq = D // P
    wr = _prep_w(w)
    br = b.reshape(Cout, 1)

    xspec = lambda fn: pl.BlockSpec((1, 2, H, C, W), fn)
    in_specs = [xspec(lambda n, d: (n, jnp.maximum(G * d - 1, 0), 0, 0, 0))]
    in_specs += [xspec(lambda n, d, j=j: (n, G * d + j, 0, 0, 0)) for j in range(G)]
    in_specs += [
        xspec(lambda n, d: (n, jnp.minimum(G * d + G, Dh - 1), 0, 0, 0)),
        pl.BlockSpec((3, 3, Cout, 3 * C), lambda n, d: (0, 0, 0, 0)),
        pl.BlockSpec((Cout, 1), lambda n, d: (0, 0)),
    ]
    args = [x] * (G + 2) + [wr, br]

    if head is not None:
        hw, hb = head
        K = hw.shape[0]
        in_specs += [
            pl.BlockSpec((K, Cout), lambda n, d: (0, 0)),
            pl.BlockSpec((K, 1), lambda n, d: (0, 0)),
        ]
        args += [hw, hb.reshape(K, 1)]
        return pl.pallas_call(
            functools.partial(_conv_head_kernel, H, Dq, P),
            out_shape=jax.ShapeDtypeStruct((N, K, D, H, W), jnp.float32),
            grid_spec=pltpu.PrefetchScalarGridSpec(
                num_scalar_prefetch=0,
                grid=(N, Dq),
                in_specs=in_specs,
                out_specs=pl.BlockSpec((1, K, P, H, W), lambda n, d: (n, 0, d, 0, 0)),
            ),
            compiler_params=_PARAMS,
        )(*args)

    if pool:
        cols = jnp.arange(W // 2)
        se = (jnp.arange(W)[:, None] == 2 * cols[None, :]).astype(_BF)
        so = (jnp.arange(W)[:, None] == 2 * cols[None, :] + 1).astype(_BF)
        in_specs += [
            pl.BlockSpec((W, W // 2), lambda n, d: (0, 0)),
            pl.BlockSpec((W, W // 2), lambda n, d: (0, 0)),
        ]
        args += [se, so]
        return pl.pallas_call(
            functools.partial(_conv_pool_kernel, H, Dq, P),
            out_shape=[
                jax.ShapeDtypeStruct((N, D, H, Cout, W), _BF),
                jax.ShapeDtypeStruct((N, D // 2, H // 2, Cout, W // 2), _BF),
            ],
            grid_spec=pltpu.PrefetchScalarGridSpec(
                num_scalar_prefetch=0,
                grid=(N, Dq),
                in_specs=in_specs,
                out_specs=[
                    pl.BlockSpec((1, P, H, Cout, W), lambda n, d: (n, d, 0, 0, 0)),
                    pl.BlockSpec((1, P // 2, H // 2, Cout, W // 2), lambda n, d: (n, d, 0, 0, 0)),
                ],
            ),
            compiler_params=_PARAMS,
        )(*args)

    return pl.pallas_call(
        functools.partial(_conv_plain_kernel, H, Dq, P),
        out_shape=jax.ShapeDtypeStruct((N, D, H, Cout, W), _BF),
        grid_spec=pltpu.PrefetchScalarGridSpec(
            num_scalar_prefetch=0,
            grid=(N, Dq),
            in_specs=in_specs,
            out_specs=pl.BlockSpec((1, P, H, Cout, W), lambda n, d: (n, d, 0, 0, 0)),
        ),
        compiler_params=_PARAMS,
    )(*args)


def _up_conv_kernel(H, Dq, P, *refs):
    nl = P // 2 + 2
    lrefs = refs[:nl]
    srefs = refs[nl:2 * nl]
    wl_ref, ws_ref, b_ref, e_ref, o_ref = refs[2 * nl:]
    m0, m2 = _halo_masks(Dq)
    E = e_ref[...]
    dims = (((2,), (0,)), ((), ()))

    def expand(p):
        return lax.dot_general(p, E, dims, preferred_element_type=jnp.float32).astype(_BF)

    f_up = [_fold_w(expand(lr[0, 0])) for lr in lrefs]
    f_up[0] = f_up[0] * m0
    f_up[-1] = f_up[-1] * m2
    f_sk = _halo_folds(srefs, Dq)
    wl = [[wl_ref[kd, kh] for kh in range(3)] for kd in range(3)]
    wsk = [[ws_ref[kd, kh] for kh in range(3)] for kd in range(3)]
    b = b_ref[...]
    accs = [[None] * H for _ in range(P)]

    def add(dp, h, t):
        accs[dp][h] = t if accs[dp][h] is None else accs[dp][h] + t

    for kd in range(3):
        for kh in range(3):
            w = wsk[kd][kh]
            for dp in range(P):
                fs = f_sk[dp + kd]
                for h in range(H):
                    r = h + kh - 1
                    if 0 <= r < H:
                        add(dp, h, jnp.dot(w, fs[r], preferred_element_type=jnp.float32))
            w = wl[kd][kh]
            cache = {}
            for dp in range(P):
                li = (dp + kd - 1) // 2 + 1
                for h in range(H):
                    r = h + kh - 1
                    if 0 <= r < H:
                        key = (li, r // 2)
                        if key not in cache:
                            cache[key] = jnp.dot(w, f_up[li][r // 2],
                                                 preferred_element_type=jnp.float32)
                        add(dp, h, cache[key])
    for dp in range(P):
        for h in range(H):
            o_ref[0, dp, h] = jnp.maximum(accs[dp][h] + b, 0.0).astype(_BF)


def _up_conv(low, skip, w, b):
    N, Dl, Hl, Cl, Wl = low.shape
    _, D, H, Cs, W = skip.shape
    Cout = w.shape[0]
    P = 8 if D % 8 == 0 else 4
    G = P // 2
    Dq = D // P
    Dh = D // 2
    wl = _prep_w(w[:, :Cl])
    wsk = _prep_w(w[:, Cl:])
    br = b.reshape(Cout, 1)
    E = (jnp.arange(Wl)[:, None] == (jnp.arange(W)[None, :] // 2)).astype(_BF)

    lspec = lambda fn: pl.BlockSpec((1, 1, Hl, Cl, Wl), fn)
    sspec = lambda fn: pl.BlockSpec((1, 2, H, Cs, W), fn)
    in_specs = [lspec(lambda n, d: (n, jnp.maximum(G * d - 1, 0), 0, 0, 0))]
    in_specs += [lspec(lambda n, d, j=j: (n, G * d + j, 0, 0, 0)) for j in range(G)]
    in_specs += [lspec(lambda n, d: (n, jnp.minimum(G * d + G, Dl - 1), 0, 0, 0))]
    in_specs += [sspec(lambda n, d: (n, jnp.maximum(G * d - 1, 0), 0, 0, 0))]
    in_specs += [sspec(lambda n, d, j=j: (n, G * d + j, 0, 0, 0)) for j in range(G)]
    in_specs += [sspec(lambda n, d: (n, jnp.minimum(G * d + G, Dh - 1), 0, 0, 0))]
    in_specs += [
        pl.BlockSpec((3, 3, Cout, 3 * Cl), lambda n, d: (0, 0, 0, 0)),
        pl.BlockSpec((3, 3, Cout, 3 * Cs), lambda n, d: (0, 0, 0, 0)),
        pl.BlockSpec((Cout, 1), lambda n, d: (0, 0)),
        pl.BlockSpec((Wl, W), lambda n, d: (0, 0)),
    ]
    nl = G + 2
    return pl.pallas_call(
        functools.partial(_up_conv_kernel, H, Dq, P),
        out_shape=jax.ShapeDtypeStruct((N, D, H, Cout, W), _BF),
        grid_spec=pltpu.PrefetchScalarGridSpec(
            num_scalar_prefetch=0,
            grid=(N, Dq),
            in_specs=in_specs,
            out_specs=pl.BlockSpec((1, P, H, Cout, W), lambda n, d: (n, d, 0, 0, 0)),
        ),
        compiler_params=_PARAMS,
    )(*([low] * nl + [skip] * nl + [wl, wsk, br, E]))


def kernel(x, w_d0c0, b_d0c0, w_d0c1, b_d0c1, w_d1c0, b_d1c0, w_d1c1, b_d1c1,
           w_d2c0, b_d2c0, w_d2c1, b_d2c1, w_u0c0, b_u0c0, w_u0c1, b_u0c1,
           w_u1c0, b_u1c0, w_u1c1, b_u1c1, head_w, head_b):
    xb = jnp.transpose(x, (0, 2, 3, 1, 4)).astype(_BF)
    h = _conv_pair(xb, w_d0c0, b_d0c0)
    s0, h = _conv_pair(h, w_d0c1, b_d0c1, pool=True)
    h = _conv_pair(h, w_d1c0, b_d1c0)
    s1, h = _conv_pair(h, w_d1c1, b_d1c1, pool=True)
    h = _conv_pair(h, w_d2c0, b_d2c0)
    h = _conv_pair(h, w_d2c1, b_d2c1)
    h = _up_conv(h, s1, w_u1c0, b_u1c0)
    h = _conv_pair(h, w_u1c1, b_u1c1)
    h = _up_conv(h, s0, w_u0c0, b_u0c0)
    return _conv_pair(h, w_u0c1, b_u0c1, head=(head_w, head_b))
